# baseline (speedup 1.0000x reference)
import functools

import jax
import jax.numpy as jnp
from jax import lax
from jax.experimental import pallas as pl
from jax.experimental.pallas import tpu as pltpu
from jax.experimental.pallas import tpu_sc as plsc

_NC = 2
_NS = 16
_NW = _NC * _NS
_LANES = 16
_CHUNK = 128


def _make_sc_call(B, D):
    assert B % _NW == 0
    bpw = B // _NW
    assert bpw % _CHUNK == 0
    nchunk = bpw // _CHUNK
    ngroups = bpw // _LANES

    mesh = plsc.VectorSubcoreMesh(core_axis_name="c", subcore_axis_name="s")

    @functools.partial(
        pl.kernel,
        out_type=[
            jax.ShapeDtypeStruct((B,), jnp.float32),
            jax.ShapeDtypeStruct((B,), jnp.float32),
        ],
        mesh=mesh,
        compiler_params=pltpu.CompilerParams(
            needs_layout_passes=False, use_tc_tiling_on_sc=False),
        scratch_types=[
            pltpu.VMEM((nchunk, _CHUNK), jnp.int32),
            pltpu.VMEM((nchunk, _CHUNK), jnp.int32),
            pltpu.VMEM((bpw, D), jnp.float32),
            pltpu.VMEM((bpw, D), jnp.float32),
            pltpu.VMEM((bpw,), jnp.float32),
            pltpu.VMEM((bpw,), jnp.float32),
            pltpu.VMEM((_LANES,), jnp.float32),
            pltpu.VMEM((bpw,), jnp.float32),
            pltpu.VMEM((bpw,), jnp.float32),
            pltpu.SemaphoreType.DMA,
        ],
    )
    def sc_call(uidx_hbm, iidx_hbm, u_hbm, v_hbm, alpha_hbm, gu_hbm, gv_hbm,
                mu_hbm, sig_hbm,
                uidx_v, iidx_v, u_rows, v_rows, gu_v, gv_v, alpha_v,
                mu_v, sig_v, sem):
        wid = lax.axis_index("s") * _NC + lax.axis_index("c")
        base = wid * bpw

        pltpu.sync_copy(uidx_hbm.at[wid], uidx_v)
        pltpu.sync_copy(iidx_hbm.at[wid], iidx_v)
        pltpu.sync_copy(alpha_hbm, alpha_v)

        copies = []
        for j in range(nchunk):
            sl = pl.ds(j * _CHUNK, _CHUNK)
            copies.append(
                pltpu.async_copy(u_hbm.at[uidx_v.at[j]], u_rows.at[sl], sem))
            copies.append(
                pltpu.async_copy(v_hbm.at[iidx_v.at[j]], v_rows.at[sl], sem))
            copies.append(
                pltpu.async_copy(gu_hbm.at[uidx_v.at[j]], gu_v.at[sl], sem))
            copies.append(
                pltpu.async_copy(gv_hbm.at[iidx_v.at[j]], gv_v.at[sl], sem))
        for c in copies:
            c.wait()

        iota = lax.iota(jnp.int32, _LANES)
        alpha = alpha_v[...]

        def body(g, carry):
            rows = g * _LANES + iota
            col = jnp.zeros((_LANES,), jnp.int32)
            acc = jnp.zeros((_LANES,), jnp.float32)
            for _ in range(D):
                un = plsc.load_gather(u_rows, [rows, col])
                vn = plsc.load_gather(v_rows, [rows, col])
                acc = acc + un * vn
                col = col + 1
            sl = pl.ds(g * _LANES, _LANES)
            mu_v[sl] = acc
            x = alpha * gu_v[sl] * gv_v[sl]
            i = plsc.bitcast(x, jnp.int32)
            i = 0x5F3759DF - lax.shift_right_logical(i, 1)
            y = plsc.bitcast(i, jnp.float32)
            for _ in range(3):
                y = y * (1.5 - 0.5 * x * y * y)
            sig_v[sl] = y
            return carry

        lax.fori_loop(0, ngroups, body, 0)

        pltpu.sync_copy(mu_v, mu_hbm.at[pl.ds(base, bpw)])
        pltpu.sync_copy(sig_v, sig_hbm.at[pl.ds(base, bpw)])

    return sc_call


def kernel(user_idx, item_idx, U, V, alpha, gamma_u, gamma_v):
    B = user_idx.shape[0]
    D = U.shape[1]
    bpw = B // _NW
    nchunk = bpw // _CHUNK
    uidx3 = user_idx.astype(jnp.int32).reshape(_NW, nchunk, _CHUNK)
    iidx3 = item_idx.astype(jnp.int32).reshape(_NW, nchunk, _CHUNK)
    alpha16 = jnp.broadcast_to(
        jnp.asarray(alpha, jnp.float32).reshape(()), (_LANES,))
    mu, sigma = _make_sc_call(B, D)(
        uidx3, iidx3, U, V, alpha16, gamma_u, gamma_v)
    return (mu, sigma)


# speedup vs baseline: 1.0028x; 1.0028x over previous
import functools

import jax
import jax.numpy as jnp
from jax import lax
from jax.experimental import pallas as pl
from jax.experimental.pallas import tpu as pltpu
from jax.experimental.pallas import tpu_sc as plsc

_NC = 2
_NS = 16
_NW = _NC * _NS
_LANES = 16
_CHUNK = 128


def _make_sc_call(B, D):
    assert B % _NW == 0
    bpw = B // _NW
    assert bpw % _CHUNK == 0
    nchunk = bpw // _CHUNK
    ngroups = bpw // _LANES

    mesh = plsc.VectorSubcoreMesh(core_axis_name="c", subcore_axis_name="s")

    @functools.partial(
        pl.kernel,
        out_type=[
            jax.ShapeDtypeStruct((B,), jnp.float32),
            jax.ShapeDtypeStruct((B,), jnp.float32),
        ],
        mesh=mesh,
        compiler_params=pltpu.CompilerParams(
            needs_layout_passes=False, use_tc_tiling_on_sc=False),
        scratch_types=[
            pltpu.VMEM((bpw,), jnp.int32),
            pltpu.VMEM((bpw,), jnp.int32),
            pltpu.VMEM((bpw, D), jnp.float32),
            pltpu.VMEM((bpw, D), jnp.float32),
            pltpu.VMEM((bpw,), jnp.float32),
            pltpu.VMEM((bpw,), jnp.float32),
            pltpu.VMEM((_LANES,), jnp.float32),
            pltpu.VMEM((bpw,), jnp.float32),
            pltpu.VMEM((bpw,), jnp.float32),
            pltpu.SemaphoreType.DMA,
        ],
    )
    def sc_call(uidx_hbm, iidx_hbm, u_hbm, v_hbm, alpha_hbm, gu_hbm, gv_hbm,
                mu_hbm, sig_hbm,
                uidx_v, iidx_v, u_rows, v_rows, gu_v, gv_v, alpha_v,
                mu_v, sig_v, sem):
        wid = lax.axis_index("s") * _NC + lax.axis_index("c")
        base = wid * bpw

        pltpu.sync_copy(uidx_hbm.at[pl.ds(base, bpw)], uidx_v)
        pltpu.sync_copy(iidx_hbm.at[pl.ds(base, bpw)], iidx_v)
        pltpu.sync_copy(alpha_hbm, alpha_v)

        copies = []
        for j in range(nchunk):
            sl = pl.ds(j * _CHUNK, _CHUNK)
            copies.append(
                pltpu.async_copy(u_hbm.at[uidx_v.at[sl]], u_rows.at[sl],
                                 sem))
            copies.append(
                pltpu.async_copy(v_hbm.at[iidx_v.at[sl]], v_rows.at[sl],
                                 sem))
            copies.append(
                pltpu.async_copy(gu_hbm.at[uidx_v.at[sl]], gu_v.at[sl],
                                 sem))
            copies.append(
                pltpu.async_copy(gv_hbm.at[iidx_v.at[sl]], gv_v.at[sl],
                                 sem))
        for c in copies:
            c.wait()

        iota = lax.iota(jnp.int32, _LANES)
        alpha = alpha_v[...]

        def body(g, carry):
            rows = g * _LANES + iota
            col = jnp.zeros((_LANES,), jnp.int32)
            acc = jnp.zeros((_LANES,), jnp.float32)
            for _ in range(D):
                un = plsc.load_gather(u_rows, [rows, col])
                vn = plsc.load_gather(v_rows, [rows, col])
                acc = acc + un * vn
                col = col + 1
            sl = pl.ds(g * _LANES, _LANES)
            mu_v[sl] = acc
            x = alpha * gu_v[sl] * gv_v[sl]
            i = plsc.bitcast(x, jnp.int32)
            i = 0x5F3759DF - lax.shift_right_logical(i, 1)
            y = plsc.bitcast(i, jnp.float32)
            for _ in range(3):
                y = y * (1.5 - 0.5 * x * y * y)
            sig_v[sl] = y
            return carry

        lax.fori_loop(0, ngroups, body, 0)

        pltpu.sync_copy(mu_v, mu_hbm.at[pl.ds(base, bpw)])
        pltpu.sync_copy(sig_v, sig_hbm.at[pl.ds(base, bpw)])

    return sc_call


def kernel(user_idx, item_idx, U, V, alpha, gamma_u, gamma_v):
    B = user_idx.shape[0]
    D = U.shape[1]
    alpha16 = jnp.broadcast_to(
        jnp.asarray(alpha, jnp.float32).reshape(()), (_LANES,))
    mu, sigma = _make_sc_call(B, D)(
        user_idx.astype(jnp.int32), item_idx.astype(jnp.int32),
        U, V, alpha16, gamma_u, gamma_v)
    return (mu, sigma)
